# baseline (speedup 1.0000x reference)
import functools

import jax
import jax.numpy as jnp
from jax import lax
from jax.experimental import pallas as pl
from jax.experimental.pallas import tpu as pltpu
from jax.experimental.pallas import tpu_sc as plsc

BATCH = 1024
SEQ = 200
EMBED = 128
HALF = SEQ // 2
NC, NS, LANES = 2, 16, 16
NW = NC * NS
ROWS_PER_W = BATCH // NW
VREGS_PER_ROW = EMBED // LANES
NBUF = 3


def _body(idx_hbm, table_hbm, pos_hbm, out_hbm,
          pos_v, idx_v, rows_v, gsem0, gsem1, gsem2, wsem0, wsem1, wsem2):
    gsem = (gsem0, gsem1, gsem2)
    wsem = (wsem0, wsem1, wsem2)
    wid = lax.axis_index("s") * NC + lax.axis_index("c")
    base = wid * ROWS_PER_W

    pltpu.sync_copy(pos_hbm, pos_v)
    pltpu.sync_copy(idx_hbm.at[pl.ds(base, ROWS_PER_W)], idx_v)

    def start_gather(b):
        buf = b % NBUF
        return [
            pltpu.async_copy(table_hbm.at[idx_v.at[b, h]],
                             rows_v.at[buf, pl.ds(h * HALF, HALF)],
                             gsem[buf])
            for h in range(2)
        ]

    def add_positions(buf):
        @plsc.parallel_loop(0, SEQ, unroll=4)
        def _(i):
            for j in range(VREGS_PER_ROW):
                sl = pl.ds(j * LANES, LANES)
                rows_v[buf, i, sl] = rows_v[buf, i, sl] + pos_v[i, sl]

    pending_g = {0: start_gather(0), 1: start_gather(1)}
    pending_w = {}
    for b in range(ROWS_PER_W):
        buf = b % NBUF
        for d in pending_g.pop(b):
            d.wait()
        if b + 2 < ROWS_PER_W:
            if b >= 1:
                pending_w.pop(b - 1).wait()
            pending_g[b + 2] = start_gather(b + 2)
        add_positions(buf)
        pending_w[b] = pltpu.async_copy(rows_v.at[buf], out_hbm.at[base + b],
                                        wsem[buf])
    for b in sorted(pending_w):
        pending_w.pop(b).wait()


@jax.jit
def _embed(idx, token_table, position_table):
    mesh = plsc.VectorSubcoreMesh(core_axis_name="c", subcore_axis_name="s",
                                  num_cores=NC, num_subcores=NS)
    run = pl.kernel(
        _body,
        out_type=jax.ShapeDtypeStruct((BATCH, SEQ, EMBED), jnp.float32),
        mesh=mesh,
        scratch_types=[
            pltpu.VMEM((SEQ, EMBED), jnp.float32),
            pltpu.VMEM((ROWS_PER_W, 2, HALF), jnp.int32),
            pltpu.VMEM((NBUF, SEQ, EMBED), jnp.float32),
            pltpu.SemaphoreType.DMA,
            pltpu.SemaphoreType.DMA,
            pltpu.SemaphoreType.DMA,
            pltpu.SemaphoreType.DMA,
            pltpu.SemaphoreType.DMA,
            pltpu.SemaphoreType.DMA,
        ],
    )
    return run(idx, token_table, position_table)


def kernel(inputs, token_table, position_table):
    idx = inputs.astype(jnp.int32).reshape(BATCH, 2, HALF)
    return _embed(idx, token_table, position_table)


# speedup vs baseline: 1.1805x; 1.1805x over previous
import functools

import jax
import jax.numpy as jnp
from jax import lax
from jax.experimental import pallas as pl
from jax.experimental.pallas import tpu as pltpu
from jax.experimental.pallas import tpu_sc as plsc

BATCH = 1024
SEQ = 200
EMBED = 128
HALF = SEQ // 2
NC, NS, LANES = 2, 16, 16
NW = NC * NS
ROWS_PER_W = BATCH // NW
VREGS_PER_ROW = EMBED // LANES
NBUF = 3


def _body(idx_hbm, table_hbm, pos_hbm, out_hbm,
          pos_v, pos_sh, idx_v, rows_v,
          gsem0, gsem1, gsem2, wsem0, wsem1, wsem2):
    gsem = (gsem0, gsem1, gsem2)
    wsem = (wsem0, wsem1, wsem2)
    sid = lax.axis_index("s")
    wid = sid * NC + lax.axis_index("c")
    base = wid * ROWS_PER_W

    pltpu.sync_copy(idx_hbm.at[pl.ds(base, ROWS_PER_W)], idx_v)

    @pl.when(sid == 0)
    def _():
        pltpu.sync_copy(pos_hbm, pos_v)
        pltpu.sync_copy(pos_v, pos_sh)

    plsc.subcore_barrier()

    def start_gather(b):
        buf = b % NBUF
        return [
            pltpu.async_copy(table_hbm.at[idx_v.at[b, h]],
                             rows_v.at[buf, pl.ds(h * HALF, HALF)],
                             gsem[buf], add=True)
            for h in range(2)
        ]

    def prefill(b):
        pltpu.sync_copy(pos_sh, rows_v.at[b % NBUF])

    prefill(0)
    prefill(1)
    pending_g = {0: start_gather(0), 1: start_gather(1)}
    pending_w = {}
    for b in range(ROWS_PER_W):
        buf = b % NBUF
        for d in pending_g.pop(b):
            d.wait()
        if b + 2 < ROWS_PER_W:
            if b >= 1:
                pending_w.pop(b - 1).wait()
            prefill(b + 2)
            pending_g[b + 2] = start_gather(b + 2)
        pending_w[b] = pltpu.async_copy(rows_v.at[buf], out_hbm.at[base + b],
                                        wsem[buf])
    for b in sorted(pending_w):
        pending_w.pop(b).wait()


@jax.jit
def _embed(idx, token_table, position_table):
    mesh = plsc.VectorSubcoreMesh(core_axis_name="c", subcore_axis_name="s",
                                  num_cores=NC, num_subcores=NS)
    run = pl.kernel(
        _body,
        out_type=jax.ShapeDtypeStruct((BATCH, SEQ, EMBED), jnp.float32),
        mesh=mesh,
        scratch_types=[
            pltpu.VMEM((SEQ, EMBED), jnp.float32),
            pltpu.VMEM_SHARED((SEQ, EMBED), jnp.float32),
            pltpu.VMEM((ROWS_PER_W, 2, HALF), jnp.int32),
            pltpu.VMEM((NBUF, SEQ, EMBED), jnp.float32),
            pltpu.SemaphoreType.DMA,
            pltpu.SemaphoreType.DMA,
            pltpu.SemaphoreType.DMA,
            pltpu.SemaphoreType.DMA,
            pltpu.SemaphoreType.DMA,
            pltpu.SemaphoreType.DMA,
        ],
    )
    return run(idx, token_table, position_table)


def kernel(inputs, token_table, position_table):
    idx = inputs.astype(jnp.int32).reshape(BATCH, 2, HALF)
    return _embed(idx, token_table, position_table)


# speedup vs baseline: 1.2702x; 1.0760x over previous
import functools

import jax
import jax.numpy as jnp
from jax import lax
from jax.experimental import pallas as pl
from jax.experimental.pallas import tpu as pltpu
from jax.experimental.pallas import tpu_sc as plsc

BATCH = 1024
SEQ = 200
EMBED = 128
HALF = SEQ // 2
NC, NS, LANES = 2, 16, 16
NW = NC * NS
ROWS_PER_W = BATCH // NW
NBUF = 4


def _body(idx_hbm, table_hbm, pos_hbm, out_hbm,
          pos_sh, idx_v, rows_v,
          gsem0, gsem1, gsem2, gsem3, wsem0, wsem1, wsem2, wsem3):
    gsem = (gsem0, gsem1, gsem2, gsem3)
    wsem = (wsem0, wsem1, wsem2, wsem3)
    sid = lax.axis_index("s")
    wid = sid * NC + lax.axis_index("c")
    base = wid * ROWS_PER_W

    pltpu.sync_copy(idx_hbm.at[pl.ds(base, ROWS_PER_W)], idx_v)

    @pl.when(sid == 0)
    def _():
        pltpu.sync_copy(pos_hbm, pos_sh)

    plsc.subcore_barrier()

    def start_gather(b):
        buf = b % NBUF
        return [
            pltpu.async_copy(table_hbm.at[idx_v.at[b, h]],
                             rows_v.at[buf, pl.ds(h * HALF, HALF)],
                             gsem[buf], add=True)
            for h in range(2)
        ]

    def prefill(b):
        pltpu.sync_copy(pos_sh, rows_v.at[b % NBUF])

    prefill(0)
    prefill(1)
    pending_g = {0: start_gather(0), 1: start_gather(1)}
    pending_w = {}
    for b in range(ROWS_PER_W):
        buf = b % NBUF
        if b + 2 < ROWS_PER_W:
            if b >= 2:
                pending_w.pop(b - 2).wait()
            prefill(b + 2)
            pending_g[b + 2] = start_gather(b + 2)
        for d in pending_g.pop(b):
            d.wait()
        pending_w[b] = pltpu.async_copy(rows_v.at[buf], out_hbm.at[base + b],
                                        wsem[buf])
    for b in sorted(pending_w):
        pending_w.pop(b).wait()


@jax.jit
def _embed(idx, token_table, position_table):
    mesh = plsc.VectorSubcoreMesh(core_axis_name="c", subcore_axis_name="s",
                                  num_cores=NC, num_subcores=NS)
    run = pl.kernel(
        _body,
        out_type=jax.ShapeDtypeStruct((BATCH, SEQ, EMBED), jnp.float32),
        mesh=mesh,
        scratch_types=[
            pltpu.VMEM_SHARED((SEQ, EMBED), jnp.float32),
            pltpu.VMEM((ROWS_PER_W, 2, HALF), jnp.int32),
            pltpu.VMEM((NBUF, SEQ, EMBED), jnp.float32),
            pltpu.SemaphoreType.DMA,
            pltpu.SemaphoreType.DMA,
            pltpu.SemaphoreType.DMA,
            pltpu.SemaphoreType.DMA,
            pltpu.SemaphoreType.DMA,
            pltpu.SemaphoreType.DMA,
            pltpu.SemaphoreType.DMA,
            pltpu.SemaphoreType.DMA,
        ],
    )
    return run(idx, token_table, position_table)


def kernel(inputs, token_table, position_table):
    idx = inputs.astype(jnp.int32).reshape(BATCH, 2, HALF)
    return _embed(idx, token_table, position_table)
